# baseline (speedup 1.0000x reference)
import dataclasses
import functools

import jax
import jax.numpy as jnp
from jax import lax
from jax.experimental import pallas as pl
from jax.experimental.pallas import tpu as pltpu
from jax.experimental.pallas import tpu_sc as plsc


def _sc_compiler_params():
    cp = pltpu.CompilerParams()
    if "needs_layout_passes" in pltpu.CompilerParams.__dataclass_fields__:
        cp = dataclasses.replace(cp, needs_layout_passes=False)
    return cp


_Q = 0.15
_TB = 2048
_CH = 16
_NW = 32
_L = 16


def _search_thresholds(scores, length, kk, t_total):
    iota = lax.broadcasted_iota(jnp.int32, scores.shape, 1)
    keys = lax.bitcast_convert_type(scores, jnp.int32)
    keys = jnp.where(iota < length, keys, jnp.int32(-1))

    def bs_body(_, lohi):
        lo, hi = lohi
        mid = lo + lax.shift_right_logical(hi - lo, 1)
        cnt_gt = jnp.sum((keys > mid).astype(jnp.int32))
        take_hi = cnt_gt >= kk
        return (jnp.where(take_hi, mid + 1, lo), jnp.where(take_hi, hi, mid))

    v, _ = lax.fori_loop(0, 32, bs_body, (jnp.int32(-1), jnp.int32(2**31 - 1)))
    cnt_gt_v = jnp.sum((keys > v).astype(jnp.int32))
    needed = kk - cnt_gt_v
    eq = keys == v

    def idx_body(_, lohi):
        lo2, hi2 = lohi
        mid = (lo2 + hi2) >> 1
        c = jnp.sum((eq & (iota <= mid)).astype(jnp.int32))
        ok = c >= needed
        return (jnp.where(ok, lo2, mid + 1), jnp.where(ok, mid, hi2))

    m, _ = lax.fori_loop(
        0, 13, idx_body, (jnp.int32(0), jnp.int32(t_total - 1))
    )
    return v, m


def _scores_body(len_ref, kv_ref, h_ref, s_ref, thr_ref, ss_ref):
    b = pl.program_id(0)
    t = pl.program_id(1)
    nt = pl.num_programs(1)
    t_total = ss_ref.shape[1]

    x = h_ref[0]
    sc = jnp.sum(x * x, axis=1)[None, :]
    s_ref[0] = sc
    ss_ref[:, pl.ds(t * _TB, _TB)] = sc

    @pl.when(t == nt - 1)
    def _():
        v, m = _search_thresholds(ss_ref[...], len_ref[b], kv_ref[b], t_total)
        thr_ref[0, 0, 0] = v
        thr_ref[0, 0, 1] = m
        thr_ref[0, 0, 2] = kv_ref[b]
        thr_ref[0, 0, 3] = len_ref[b]
        inv = jnp.float32(1.0) / kv_ref[b].astype(jnp.float32)
        thr_ref[0, 0, 4] = lax.bitcast_convert_type(inv, jnp.int32)


def _sc_pool_body(
    scores_hbm, thr_hbm, hv_hbm, out_hbm, scores_v, thr_v, idx_v,
    buf0, buf1, acc, comb, res_v, stage, sem0, sem1
):
    nw, nb, tw = scores_hbm.shape
    d = acc.shape[1]
    dq = d // (16 // nb)
    t_total = hv_hbm.shape[0] // nb
    shift = t_total.bit_length() - 1
    core = lax.axis_index("c")
    sub = lax.axis_index("s")
    wid = sub * 2 + core
    per_w = tw // _L

    pltpu.sync_copy(scores_hbm.at[wid], scores_v)
    pltpu.sync_copy(thr_hbm.at[:], thr_v)

    lanes = lax.iota(jnp.int32, _L)

    def _extract(e):
        return jnp.sum(jnp.where(lanes == (e % _L), thr_v[(e // _L)], 0))

    zero_v = jnp.zeros((_L,), jnp.float32)
    for r in range(nb):
        for p in range(0, d, _L):
            acc[r, pl.ds(p, _L)] = zero_v

    zero_i = jnp.zeros((_L,), jnp.int32)
    for i in range(0, idx_v.shape[0], _L):
        idx_v[pl.ds(i, _L)] = zero_i

    cnt = jnp.int32(0)
    for b in range(nb):
        v = _extract(b * _L + 0)
        m = _extract(b * _L + 1)
        ln = _extract(b * _L + 3)

        def chunk(i, c, b=b, v=v, m=m, ln=ln):
            sv = scores_v[b, pl.ds(i * _L, _L)]
            keys = plsc.bitcast(sv, jnp.int32)
            tv = lanes + (wid + i * _NW) * _L
            keys = jnp.where(tv < ln, keys, jnp.int32(-1))
            selm = (keys > v) | ((keys == v) & (tv <= m))
            gidx = b * t_total + tv
            plsc.store_compressed(idx_v.at[pl.ds(c, _L)], gidx, mask=selm)
            return c + jnp.sum(selm.astype(jnp.int32))

        cnt = lax.fori_loop(0, per_w, chunk, cnt)

    nch = (cnt + _CH - 1) // _CH
    npair = (nch + 1) // 2

    def accum(buf, ch):
        idxs = idx_v[pl.ds(ch * _CH, _CH)]
        tags = lax.shift_right_logical(idxs, shift)

        def row(r, c2, tags=tags, ch=ch):
            @pl.when(ch * _CH + r < cnt)
            def _():
                tg = jnp.sum(jnp.where(lanes == r, tags, 0))
                for p in range(0, d, _L):
                    plsc.addupdate(
                        acc.at[tg, pl.ds(p, _L)], buf[r, pl.ds(p, _L)]
                    )

            return c2

        lax.fori_loop(0, _CH, row, 0)

    def pair(i, carry):
        ch0 = 2 * i
        ch1 = 2 * i + 1
        cp0 = pltpu.async_copy(
            hv_hbm.at[idx_v.at[pl.ds(ch0 * _CH, _CH)]], buf0, sem0
        )

        @pl.when(ch1 < nch)
        def _():
            pltpu.async_copy(
                hv_hbm.at[idx_v.at[pl.ds(ch1 * _CH, _CH)]], buf1, sem1
            )

        cp0.wait()
        accum(buf0, ch0)

        @pl.when(ch1 < nch)
        def _():
            pltpu.make_async_copy(
                hv_hbm.at[idx_v.at[pl.ds(ch1 * _CH, _CH)]], buf1, sem1
            ).wait()

        accum(buf1, ch1)
        return carry

    lax.fori_loop(0, npair, pair, 0)

    for b in range(nb):
        pltpu.sync_copy(acc.at[pl.ds(b, 1)], stage.at[pl.ds(b * 16 + sub, 1)])

    plsc.subcore_barrier()

    nq = 16 // nb
    bq = sub // nq
    qd = sub % nq
    pltpu.sync_copy(stage.at[pl.ds(bq * 16, 16), pl.ds(qd * dq, dq)], comb)
    invv = plsc.bitcast(
        jnp.broadcast_to(_extract(bq * _L + 4), (_L,)), jnp.float32
    )

    for p in range(0, dq, _L):
        s0 = comb[0, pl.ds(p, _L)]
        for w in range(1, 16):
            s0 = s0 + comb[w, pl.ds(p, _L)]
        res_v[0, pl.ds(p, _L)] = s0 * invv

    pltpu.sync_copy(res_v, out_hbm.at[core, pl.ds(bq, 1), pl.ds(qd * dq, dq)])


def _pipeline(H, lengths, kv):
    B, T, D = H.shape
    assert 16 % B == 0 and T % (_L * _NW) == 0 and D % (16 // B * _L) == 0

    scores, thr = pl.pallas_call(
        _scores_body,
        grid=(B, T // _TB),
        in_specs=[
            pl.BlockSpec(memory_space=pltpu.SMEM),
            pl.BlockSpec(memory_space=pltpu.SMEM),
            pl.BlockSpec((1, _TB, D), lambda b, t: (b, t, 0)),
        ],
        out_specs=[
            pl.BlockSpec((1, 1, _TB), lambda b, t: (b, 0, t)),
            pl.BlockSpec(
                (1, 1, 16), lambda b, t: (b, 0, 0), memory_space=pltpu.SMEM
            ),
        ],
        out_shape=[
            jax.ShapeDtypeStruct((B, 1, T), jnp.float32),
            jax.ShapeDtypeStruct((B, 1, 16), jnp.int32),
        ],
        scratch_shapes=[pltpu.VMEM((1, T), jnp.float32)],
    )(lengths, kv, H)

    mesh = plsc.VectorSubcoreMesh(core_axis_name="c", subcore_axis_name="s")
    cap = B * (T // _NW) + _CH

    sc_fn = pl.kernel(
        _sc_pool_body,
        out_type=jax.ShapeDtypeStruct((2, B, D), jnp.float32),
        mesh=mesh,
        scratch_types=[
            pltpu.VMEM((B, T // _NW), jnp.float32),
            pltpu.VMEM((B, 16), jnp.int32),
            pltpu.VMEM((cap,), jnp.int32),
            pltpu.VMEM((_CH, D), jnp.float32),
            pltpu.VMEM((_CH, D), jnp.float32),
            pltpu.VMEM((B, D), jnp.float32),
            pltpu.VMEM((16, D // (16 // B)), jnp.float32),
            pltpu.VMEM((1, D // (16 // B)), jnp.float32),
            pltpu.VMEM_SHARED((B * 16, D), jnp.float32),
            pltpu.SemaphoreType.DMA,
            pltpu.SemaphoreType.DMA,
        ],
        compiler_params=_sc_compiler_params(),
    )

    scores_w = (
        scores.reshape(B, T // (_NW * _L), _NW, _L)
        .transpose(2, 0, 1, 3)
        .reshape(_NW, B, T // _NW)
    )
    parts = sc_fn(scores_w, thr.reshape(B, 16), H.reshape(B * T, D))
    return parts[0] + parts[1]


_SPLIT = 2


def kernel(H, lengths):
    B, T, D = H.shape
    lengths = lengths.astype(jnp.int32)
    kv = jnp.maximum(
        jnp.ceil(lengths.astype(jnp.float32) * _Q).astype(jnp.int32), 1
    )
    nbs = B // _SPLIT
    outs = [
        _pipeline(
            H[i * nbs:(i + 1) * nbs],
            lengths[i * nbs:(i + 1) * nbs],
            kv[i * nbs:(i + 1) * nbs],
        )
        for i in range(_SPLIT)
    ]
    return jnp.concatenate(outs, axis=0) if len(outs) > 1 else outs[0]


# speedup vs baseline: 1.8224x; 1.8224x over previous
import dataclasses
import functools

import jax
import jax.numpy as jnp
from jax import lax
from jax.experimental import pallas as pl
from jax.experimental.pallas import tpu as pltpu
from jax.experimental.pallas import tpu_sc as plsc


def _sc_compiler_params():
    cp = pltpu.CompilerParams()
    if "needs_layout_passes" in pltpu.CompilerParams.__dataclass_fields__:
        cp = dataclasses.replace(cp, needs_layout_passes=False)
    return cp


_Q = 0.15
_TB = 2048
_CH = 16
_NW = 32
_L = 16


def _search_thresholds(scores, length, kk, t_total):
    iota = lax.broadcasted_iota(jnp.int32, scores.shape, 1)
    keys = lax.bitcast_convert_type(scores, jnp.int32)
    keys = jnp.where(iota < length, keys, jnp.int32(-1))

    def bs_body(_, lohi):
        lo, hi = lohi
        mid = lo + lax.shift_right_logical(hi - lo, 1)
        cnt_gt = jnp.sum((keys > mid).astype(jnp.int32))
        take_hi = cnt_gt >= kk
        return (jnp.where(take_hi, mid + 1, lo), jnp.where(take_hi, hi, mid))

    v, _ = lax.fori_loop(0, 32, bs_body, (jnp.int32(-1), jnp.int32(2**31 - 1)))
    cnt_gt_v = jnp.sum((keys > v).astype(jnp.int32))
    needed = kk - cnt_gt_v
    eq = keys == v

    def idx_body(_, lohi):
        lo2, hi2 = lohi
        mid = (lo2 + hi2) >> 1
        c = jnp.sum((eq & (iota <= mid)).astype(jnp.int32))
        ok = c >= needed
        return (jnp.where(ok, lo2, mid + 1), jnp.where(ok, mid, hi2))

    m, _ = lax.fori_loop(
        0, 13, idx_body, (jnp.int32(0), jnp.int32(t_total - 1))
    )
    return v, m


def _scores_body(len_ref, kv_ref, h_ref, s_ref, thr_ref, ss_ref):
    b = pl.program_id(0)
    t = pl.program_id(1)
    nt = pl.num_programs(1)
    t_total = ss_ref.shape[1]

    x = h_ref[0]
    sc = jnp.sum(x * x, axis=1)[None, :]
    s_ref[0] = sc
    ss_ref[:, pl.ds(t * _TB, _TB)] = sc

    @pl.when(t == nt - 1)
    def _():
        v, m = _search_thresholds(ss_ref[...], len_ref[b], kv_ref[b], t_total)
        thr_ref[0, 0, 0] = v
        thr_ref[0, 0, 1] = m
        thr_ref[0, 0, 2] = kv_ref[b]
        thr_ref[0, 0, 3] = len_ref[b]
        inv = jnp.float32(1.0) / kv_ref[b].astype(jnp.float32)
        thr_ref[0, 0, 4] = lax.bitcast_convert_type(inv, jnp.int32)


def _sc_pool_body(
    scores_hbm, thr_hbm, hv_hbm, out_hbm, scores_v, thr_v, idx_v,
    buf0, buf1, acc, comb, res_v, stage, sem0, sem1
):
    nw, nb, tw = scores_hbm.shape
    d = acc.shape[1]
    dq = d // (16 // nb)
    t_total = hv_hbm.shape[0] // nb
    shift = t_total.bit_length() - 1
    core = lax.axis_index("c")
    sub = lax.axis_index("s")
    wid = sub * 2 + core
    per_w = tw // _L

    pltpu.sync_copy(scores_hbm.at[wid], scores_v)
    pltpu.sync_copy(thr_hbm.at[:], thr_v)

    lanes = lax.iota(jnp.int32, _L)

    def _extract(e):
        return jnp.sum(jnp.where(lanes == (e % _L), thr_v[(e // _L)], 0))

    zero_v = jnp.zeros((_L,), jnp.float32)
    for r in range(nb):
        for p in range(0, d, _L):
            acc[r, pl.ds(p, _L)] = zero_v

    zero_i = jnp.zeros((_L,), jnp.int32)
    for i in range(0, idx_v.shape[0], _L):
        idx_v[pl.ds(i, _L)] = zero_i

    cnt = jnp.int32(0)
    for b in range(nb):
        v = _extract(b * _L + 0)
        m = _extract(b * _L + 1)
        ln = _extract(b * _L + 3)

        def chunk(i, c, b=b, v=v, m=m, ln=ln):
            sv = scores_v[b, pl.ds(i * _L, _L)]
            keys = plsc.bitcast(sv, jnp.int32)
            tv = lanes + (wid + i * _NW) * _L
            keys = jnp.where(tv < ln, keys, jnp.int32(-1))
            selm = (keys > v) | ((keys == v) & (tv <= m))
            gidx = b * t_total + tv
            plsc.store_compressed(idx_v.at[pl.ds(c, _L)], gidx, mask=selm)
            return c + jnp.sum(selm.astype(jnp.int32))

        cnt = lax.fori_loop(0, per_w, chunk, cnt)

    nch = (cnt + _CH - 1) // _CH
    npair = (nch + 1) // 2

    def accum(buf, ch):
        idxs = idx_v[pl.ds(ch * _CH, _CH)]
        tags = lax.shift_right_logical(idxs, shift)

        def row(r, c2, tags=tags, ch=ch):
            @pl.when(ch * _CH + r < cnt)
            def _():
                tg = jnp.sum(jnp.where(lanes == r, tags, 0))
                for p in range(0, d, _L):
                    plsc.addupdate(
                        acc.at[tg, pl.ds(p, _L)], buf[r, pl.ds(p, _L)]
                    )

            return c2

        lax.fori_loop(0, _CH, row, 0)

    def pair(i, carry):
        ch0 = 2 * i
        ch1 = 2 * i + 1
        cp0 = pltpu.async_copy(
            hv_hbm.at[idx_v.at[pl.ds(ch0 * _CH, _CH)]], buf0, sem0
        )

        @pl.when(ch1 < nch)
        def _():
            pltpu.async_copy(
                hv_hbm.at[idx_v.at[pl.ds(ch1 * _CH, _CH)]], buf1, sem1
            )

        cp0.wait()
        accum(buf0, ch0)

        @pl.when(ch1 < nch)
        def _():
            pltpu.make_async_copy(
                hv_hbm.at[idx_v.at[pl.ds(ch1 * _CH, _CH)]], buf1, sem1
            ).wait()

        accum(buf1, ch1)
        return carry

    lax.fori_loop(0, npair, pair, 0)

    for b in range(nb):
        pltpu.sync_copy(acc.at[pl.ds(b, 1)], stage.at[pl.ds(b * 16 + sub, 1)])

    plsc.subcore_barrier()

    nq = 16 // nb
    bq = sub // nq
    qd = sub % nq
    pltpu.sync_copy(stage.at[pl.ds(bq * 16, 16), pl.ds(qd * dq, dq)], comb)
    invv = plsc.bitcast(
        jnp.broadcast_to(_extract(bq * _L + 4), (_L,)), jnp.float32
    )

    for p in range(0, dq, _L):
        s0 = comb[0, pl.ds(p, _L)]
        for w in range(1, 16):
            s0 = s0 + comb[w, pl.ds(p, _L)]
        res_v[0, pl.ds(p, _L)] = s0 * invv

    pltpu.sync_copy(res_v, out_hbm.at[core, pl.ds(bq, 1), pl.ds(qd * dq, dq)])


def _pipeline(H, lengths, kv):
    B, T, D = H.shape
    assert 16 % B == 0 and T % (_L * _NW) == 0 and D % (16 // B * _L) == 0

    scores, thr = pl.pallas_call(
        _scores_body,
        grid=(B, T // _TB),
        in_specs=[
            pl.BlockSpec(memory_space=pltpu.SMEM),
            pl.BlockSpec(memory_space=pltpu.SMEM),
            pl.BlockSpec((1, _TB, D), lambda b, t: (b, t, 0)),
        ],
        out_specs=[
            pl.BlockSpec((1, 1, _TB), lambda b, t: (b, 0, t)),
            pl.BlockSpec(
                (1, 1, 16), lambda b, t: (b, 0, 0), memory_space=pltpu.SMEM
            ),
        ],
        out_shape=[
            jax.ShapeDtypeStruct((B, 1, T), jnp.float32),
            jax.ShapeDtypeStruct((B, 1, 16), jnp.int32),
        ],
        scratch_shapes=[pltpu.VMEM((1, T), jnp.float32)],
    )(lengths, kv, H)

    mesh = plsc.VectorSubcoreMesh(core_axis_name="c", subcore_axis_name="s")
    cap = B * (T // _NW) + _CH

    sc_fn = pl.kernel(
        _sc_pool_body,
        out_type=jax.ShapeDtypeStruct((2, B, D), jnp.float32),
        mesh=mesh,
        scratch_types=[
            pltpu.VMEM((B, T // _NW), jnp.float32),
            pltpu.VMEM((B, 16), jnp.int32),
            pltpu.VMEM((cap,), jnp.int32),
            pltpu.VMEM((_CH, D), jnp.float32),
            pltpu.VMEM((_CH, D), jnp.float32),
            pltpu.VMEM((B, D), jnp.float32),
            pltpu.VMEM((16, D // (16 // B)), jnp.float32),
            pltpu.VMEM((1, D // (16 // B)), jnp.float32),
            pltpu.VMEM_SHARED((B * 16, D), jnp.float32),
            pltpu.SemaphoreType.DMA,
            pltpu.SemaphoreType.DMA,
        ],
        compiler_params=_sc_compiler_params(),
    )

    scores_w = (
        scores.reshape(B, T // (_NW * _L), _NW, _L)
        .transpose(2, 0, 1, 3)
        .reshape(_NW, B, T // _NW)
    )
    parts = sc_fn(scores_w, thr.reshape(B, 16), H.reshape(B * T, D))
    return parts[0] + parts[1]


_SPLIT = 1


def kernel(H, lengths):
    B, T, D = H.shape
    lengths = lengths.astype(jnp.int32)
    kv = jnp.maximum(
        jnp.ceil(lengths.astype(jnp.float32) * _Q).astype(jnp.int32), 1
    )
    nbs = B // _SPLIT
    outs = [
        _pipeline(
            H[i * nbs:(i + 1) * nbs],
            lengths[i * nbs:(i + 1) * nbs],
            kv[i * nbs:(i + 1) * nbs],
        )
        for i in range(_SPLIT)
    ]
    return jnp.concatenate(outs, axis=0) if len(outs) > 1 else outs[0]


# speedup vs baseline: 2.1143x; 1.1602x over previous
import dataclasses
import functools

import jax
import jax.numpy as jnp
from jax import lax
from jax.experimental import pallas as pl
from jax.experimental.pallas import tpu as pltpu
from jax.experimental.pallas import tpu_sc as plsc


def _sc_compiler_params():
    cp = pltpu.CompilerParams()
    if "needs_layout_passes" in pltpu.CompilerParams.__dataclass_fields__:
        cp = dataclasses.replace(cp, needs_layout_passes=False)
    return cp


_Q = 0.15
_TB = 2048
_CH = 16
_NW = 32
_L = 16


def _scores_body(len_ref, kv_ref, h_ref, s_ref, thr_ref, ss_ref):
    b = pl.program_id(0)
    t = pl.program_id(1)
    nb_ = pl.num_programs(0)
    nt = pl.num_programs(1)
    nb, t_total = ss_ref.shape

    x = h_ref[0]
    sc = jnp.sum(x * x, axis=1)[None, :]
    s_ref[0] = sc
    ss_ref[pl.ds(b, 1), pl.ds(t * _TB, _TB)] = sc

    @pl.when((b == nb_ - 1) & (t == nt - 1))
    def _():
        rowio = lax.broadcasted_iota(jnp.int32, (nb, 1), 0)
        colio = lax.broadcasted_iota(jnp.int32, (nb, t_total), 1)
        len_v = jnp.zeros((nb, 1), jnp.int32)
        kv_v = jnp.zeros((nb, 1), jnp.int32)
        for bb in range(nb):
            len_v = jnp.where(rowio == bb, len_ref[bb], len_v)
            kv_v = jnp.where(rowio == bb, kv_ref[bb], kv_v)

        keys = lax.bitcast_convert_type(ss_ref[...], jnp.int32)
        keys = jnp.where(colio < len_v, keys, jnp.int32(-1))

        def bs_body(_, lohi):
            lo, hi = lohi
            mid = lo + lax.shift_right_logical(hi - lo, 1)
            cnt = jnp.sum((keys > mid).astype(jnp.int32), 1, keepdims=True)
            take = cnt >= kv_v
            return (jnp.where(take, mid + 1, lo), jnp.where(take, hi, mid))

        init = (
            jnp.full((nb, 1), -1, jnp.int32),
            jnp.full((nb, 1), 2**31 - 1, jnp.int32),
        )
        v_col, _ = lax.fori_loop(0, 32, bs_body, init)
        cnt_gt = jnp.sum((keys > v_col).astype(jnp.int32), 1, keepdims=True)
        needed = kv_v - cnt_gt
        eq = keys == v_col

        def idx_body(_, lohi):
            lo2, hi2 = lohi
            mid = (lo2 + hi2) >> 1
            c = jnp.sum(
                (eq & (colio <= mid)).astype(jnp.int32), 1, keepdims=True
            )
            ok = c >= needed
            return (jnp.where(ok, lo2, mid + 1), jnp.where(ok, mid, hi2))

        init2 = (
            jnp.zeros((nb, 1), jnp.int32),
            jnp.full((nb, 1), t_total - 1, jnp.int32),
        )
        m_col, _ = lax.fori_loop(0, 13, idx_body, init2)

        for bb in range(nb):
            sel = rowio == bb
            thr_ref[bb, 0, 0] = jnp.sum(jnp.where(sel, v_col, 0))
            thr_ref[bb, 0, 1] = jnp.sum(jnp.where(sel, m_col, 0))
            thr_ref[bb, 0, 2] = kv_ref[bb]
            thr_ref[bb, 0, 3] = len_ref[bb]
            inv = jnp.float32(1.0) / kv_ref[bb].astype(jnp.float32)
            thr_ref[bb, 0, 4] = lax.bitcast_convert_type(inv, jnp.int32)


def _sc_pool_body(
    scores_hbm, thr_hbm, hv_hbm, out_hbm, scores_v, thr_v, idx_v,
    buf0, buf1, acc, comb, res_v, stage, sem0, sem1
):
    nw, nb, tw = scores_hbm.shape
    d = acc.shape[1]
    dq = d // (16 // nb)
    t_total = hv_hbm.shape[0] // nb
    shift = t_total.bit_length() - 1
    core = lax.axis_index("c")
    sub = lax.axis_index("s")
    wid = sub * 2 + core
    per_w = tw // _L

    pltpu.sync_copy(scores_hbm.at[wid], scores_v)
    pltpu.sync_copy(thr_hbm.at[:], thr_v)

    lanes = lax.iota(jnp.int32, _L)

    def _extract(e):
        return jnp.sum(jnp.where(lanes == (e % _L), thr_v[(e // _L)], 0))

    zero_v = jnp.zeros((_L,), jnp.float32)
    for r in range(nb):
        for p in range(0, d, _L):
            acc[r, pl.ds(p, _L)] = zero_v

    zero_i = jnp.zeros((_L,), jnp.int32)
    for i in range(0, idx_v.shape[0], _L):
        idx_v[pl.ds(i, _L)] = zero_i

    cnt = jnp.int32(0)
    for b in range(nb):
        v = _extract(b * _L + 0)
        m = _extract(b * _L + 1)
        ln = _extract(b * _L + 3)

        def chunk(i, c, b=b, v=v, m=m, ln=ln):
            sv = scores_v[b, pl.ds(i * _L, _L)]
            keys = plsc.bitcast(sv, jnp.int32)
            tv = lanes + (wid + i * _NW) * _L
            keys = jnp.where(tv < ln, keys, jnp.int32(-1))
            selm = (keys > v) | ((keys == v) & (tv <= m))
            gidx = b * t_total + tv
            plsc.store_compressed(idx_v.at[pl.ds(c, _L)], gidx, mask=selm)
            return c + jnp.sum(selm.astype(jnp.int32))

        cnt = lax.fori_loop(0, per_w, chunk, cnt)

    nch = (cnt + _CH - 1) // _CH
    npair = (nch + 1) // 2

    def accum(buf, ch):
        idxs = idx_v[pl.ds(ch * _CH, _CH)]
        tags = lax.shift_right_logical(idxs, shift)

        def row(r, c2, tags=tags, ch=ch):
            @pl.when(ch * _CH + r < cnt)
            def _():
                tg = jnp.sum(jnp.where(lanes == r, tags, 0))
                for p in range(0, d, _L):
                    plsc.addupdate(
                        acc.at[tg, pl.ds(p, _L)], buf[r, pl.ds(p, _L)]
                    )

            return c2

        lax.fori_loop(0, _CH, row, 0)

    def pair(i, carry):
        ch0 = 2 * i
        ch1 = 2 * i + 1
        cp0 = pltpu.async_copy(
            hv_hbm.at[idx_v.at[pl.ds(ch0 * _CH, _CH)]], buf0, sem0
        )

        @pl.when(ch1 < nch)
        def _():
            pltpu.async_copy(
                hv_hbm.at[idx_v.at[pl.ds(ch1 * _CH, _CH)]], buf1, sem1
            )

        cp0.wait()
        accum(buf0, ch0)

        @pl.when(ch1 < nch)
        def _():
            pltpu.make_async_copy(
                hv_hbm.at[idx_v.at[pl.ds(ch1 * _CH, _CH)]], buf1, sem1
            ).wait()

        accum(buf1, ch1)
        return carry

    lax.fori_loop(0, npair, pair, 0)

    for b in range(nb):
        pltpu.sync_copy(acc.at[pl.ds(b, 1)], stage.at[pl.ds(b * 16 + sub, 1)])

    plsc.subcore_barrier()

    nq = 16 // nb
    bq = sub // nq
    qd = sub % nq
    pltpu.sync_copy(stage.at[pl.ds(bq * 16, 16), pl.ds(qd * dq, dq)], comb)
    invv = plsc.bitcast(
        jnp.broadcast_to(_extract(bq * _L + 4), (_L,)), jnp.float32
    )

    for p in range(0, dq, _L):
        s0 = comb[0, pl.ds(p, _L)]
        for w in range(1, 16):
            s0 = s0 + comb[w, pl.ds(p, _L)]
        res_v[0, pl.ds(p, _L)] = s0 * invv

    pltpu.sync_copy(res_v, out_hbm.at[core, pl.ds(bq, 1), pl.ds(qd * dq, dq)])


def _pipeline(H, lengths, kv):
    B, T, D = H.shape
    assert 16 % B == 0 and T % (_L * _NW) == 0 and D % (16 // B * _L) == 0

    scores, thr = pl.pallas_call(
        _scores_body,
        grid=(B, T // _TB),
        in_specs=[
            pl.BlockSpec(memory_space=pltpu.SMEM),
            pl.BlockSpec(memory_space=pltpu.SMEM),
            pl.BlockSpec((1, _TB, D), lambda b, t: (b, t, 0)),
        ],
        out_specs=[
            pl.BlockSpec((1, 1, _TB), lambda b, t: (b, 0, t)),
            pl.BlockSpec(
                (B, 1, 16), lambda b, t: (0, 0, 0), memory_space=pltpu.SMEM
            ),
        ],
        out_shape=[
            jax.ShapeDtypeStruct((B, 1, T), jnp.float32),
            jax.ShapeDtypeStruct((B, 1, 16), jnp.int32),
        ],
        scratch_shapes=[pltpu.VMEM((B, T), jnp.float32)],
    )(lengths, kv, H)

    mesh = plsc.VectorSubcoreMesh(core_axis_name="c", subcore_axis_name="s")
    cap = B * (T // _NW) + _CH

    sc_fn = pl.kernel(
        _sc_pool_body,
        out_type=jax.ShapeDtypeStruct((2, B, D), jnp.float32),
        mesh=mesh,
        scratch_types=[
            pltpu.VMEM((B, T // _NW), jnp.float32),
            pltpu.VMEM((B, 16), jnp.int32),
            pltpu.VMEM((cap,), jnp.int32),
            pltpu.VMEM((_CH, D), jnp.float32),
            pltpu.VMEM((_CH, D), jnp.float32),
            pltpu.VMEM((B, D), jnp.float32),
            pltpu.VMEM((16, D // (16 // B)), jnp.float32),
            pltpu.VMEM((1, D // (16 // B)), jnp.float32),
            pltpu.VMEM_SHARED((B * 16, D), jnp.float32),
            pltpu.SemaphoreType.DMA,
            pltpu.SemaphoreType.DMA,
        ],
        compiler_params=_sc_compiler_params(),
    )

    scores_w = (
        scores.reshape(B, T // (_NW * _L), _NW, _L)
        .transpose(2, 0, 1, 3)
        .reshape(_NW, B, T // _NW)
    )
    parts = sc_fn(scores_w, thr.reshape(B, 16), H.reshape(B * T, D))
    return parts[0] + parts[1]


_SPLIT = 1


def kernel(H, lengths):
    B, T, D = H.shape
    lengths = lengths.astype(jnp.int32)
    kv = jnp.maximum(
        jnp.ceil(lengths.astype(jnp.float32) * _Q).astype(jnp.int32), 1
    )
    nbs = B // _SPLIT
    outs = [
        _pipeline(
            H[i * nbs:(i + 1) * nbs],
            lengths[i * nbs:(i + 1) * nbs],
            kv[i * nbs:(i + 1) * nbs],
        )
        for i in range(_SPLIT)
    ]
    return jnp.concatenate(outs, axis=0) if len(outs) > 1 else outs[0]
